# Initial kernel scaffold; baseline (speedup 1.0000x reference)
import jax
import jax.numpy as jnp
from jax.experimental import pallas as pl


def kernel(x, emb_table, bot_w0, bot_b0, bot_w1, bot_b1, bot_w2, bot_b2, top_w0, top_b0, top_w1, top_b1, top_w2, top_b2, top_w3, top_b3, top_w4, top_b4):
    raise NotImplementedError("write your pallas kernel here")


# speedup vs baseline: 11.8618x; 11.8618x over previous
import functools

import jax
import jax.numpy as jnp
from jax import lax
from jax.experimental import pallas as pl
from jax.experimental.pallas import tpu as pltpu
from jax.experimental.pallas import tpu_sc as plsc

_VOCAB = 1000000
_EMB = 128
_B = 4096
_NDENSE = 13
_NSPARSE = 26
_BOT = 256

_NC, _NS = 2, 16
_NW = _NC * _NS
_TOT = _B * _NSPARSE
_PERW = _TOT // _NW
_CHUNK = 128
_NCHUNK = _PERW // _CHUNK


def _sc_gather(table, xs_flat):
  mesh = plsc.VectorSubcoreMesh(
      core_axis_name="c", subcore_axis_name="s",
      num_cores=_NC, num_subcores=_NS)

  @functools.partial(
      pl.kernel,
      out_type=jax.ShapeDtypeStruct((_TOT, _EMB), jnp.float32),
      mesh=mesh,
      scratch_types=[
          pltpu.VMEM((_PERW,), jnp.float32),
          pltpu.VMEM((_NCHUNK, _CHUNK), jnp.int32),
          pltpu.VMEM((2, _CHUNK, _EMB), jnp.float32),
          pltpu.SemaphoreType.DMA,
          pltpu.SemaphoreType.DMA,
          pltpu.SemaphoreType.DMA,
          pltpu.SemaphoreType.DMA,
      ],
  )
  def gather_kernel(table_hbm, xs_hbm, out_hbm, xv, idxv, bufs,
                    gs0, gs1, cs0, cs1):
    wid = lax.axis_index("s") * _NC + lax.axis_index("c")
    base = wid * _PERW
    pltpu.sync_copy(xs_hbm.at[pl.ds(base, _PERW)], xv)
    gsems = (gs0, gs1)
    csems = (cs0, cs1)

    def conv(j, carry):
      for i in range(_CHUNK // 16):
        v = xv[pl.ds(j * _CHUNK + i * 16, 16)]
        idxv[j, pl.ds(i * 16, 16)] = v.astype(jnp.int32) % _VOCAB
      return carry

    lax.fori_loop(0, _NCHUNK, conv, 0)

    def chunk_step(j, b, wait_prev):
      if wait_prev:
        pltpu.make_async_copy(
            bufs.at[b],
            out_hbm.at[pl.ds(base + (j - 2) * _CHUNK, _CHUNK)],
            csems[b]).wait()
      pltpu.async_copy(table_hbm.at[idxv.at[j]], bufs.at[b], gsems[b]).wait()
      pltpu.async_copy(
          bufs.at[b], out_hbm.at[pl.ds(base + j * _CHUNK, _CHUNK)], csems[b])

    for b in range(2):
      chunk_step(b, b, False)

    def pair(i, carry):
      for b in range(2):
        chunk_step(2 * i + b, b, True)
      return carry

    lax.fori_loop(1, _NCHUNK // 2, pair, 0)

    for b in range(2):
      pltpu.make_async_copy(
          bufs.at[b],
          out_hbm.at[pl.ds(base + (_NCHUNK - 2 + b) * _CHUNK, _CHUNK)],
          csems[b]).wait()

  return gather_kernel(table, xs_flat)


def _mlp_body(dense_ref, emb_ref,
              bw0, bb0, bw1, bb1, bw2, bb2,
              tw0, tb0, tw1, tb1, tw2, tb2, tw3, tb3, tw4, tb4,
              out_ref):
  f32 = jnp.float32
  dense = dense_ref[...]
  h = jax.nn.relu(jnp.dot(dense, bw0[...], preferred_element_type=f32)
                  + bb0[...])
  h = jax.nn.relu(jnp.dot(h, bw1[...], preferred_element_type=f32)
                  + bb1[...]) + h
  h = jax.nn.relu(jnp.dot(h, bw2[...], preferred_element_type=f32)
                  + bb2[...]) + h
  t = (jnp.dot(h, tw0[:_BOT, :], preferred_element_type=f32)
       + jnp.dot(emb_ref[...], tw0[_BOT:, :], preferred_element_type=f32)
       + tb0[...])
  t = jax.nn.relu(t)
  t = jax.nn.relu(jnp.dot(t, tw1[...], preferred_element_type=f32)
                  + tb1[...]) + t
  t = jax.nn.relu(jnp.dot(t, tw2[...], preferred_element_type=f32)
                  + tb2[...]) + t
  t = jax.nn.relu(jnp.dot(t, tw3[...], preferred_element_type=f32)
                  + tb3[...]) + t
  out_ref[...] = jnp.dot(t, tw4[...], preferred_element_type=f32) + tb4[...]


def _tc_mlp(dense, emb, bw0, bb0, bw1, bb1, bw2, bb2,
            tw0, tb0, tw1, tb1, tw2, tb2, tw3, tb3, tw4, tb4):
  bb = 512
  grid = (_B // bb,)

  def full(w):
    return pl.BlockSpec(w.shape, lambda i: (0,) * w.ndim)

  weights = (bw0, bb0, bw1, bb1, bw2, bb2,
             tw0, tb0, tw1, tb1, tw2, tb2, tw3, tb3, tw4, tb4)
  return pl.pallas_call(
      _mlp_body,
      grid=grid,
      in_specs=[
          pl.BlockSpec((bb, _NDENSE), lambda i: (i, 0)),
          pl.BlockSpec((bb, _NSPARSE * _EMB), lambda i: (i, 0)),
      ] + [full(w) for w in weights],
      out_specs=pl.BlockSpec((bb, 1), lambda i: (i, 0)),
      out_shape=jax.ShapeDtypeStruct((_B, 1), jnp.float32),
  )(dense, emb, *weights)


def kernel(x, emb_table, bot_w0, bot_b0, bot_w1, bot_b1, bot_w2, bot_b2,
           top_w0, top_b0, top_w1, top_b1, top_w2, top_b2,
           top_w3, top_b3, top_w4, top_b4):
  dense = x[:, :_NDENSE]
  xs_flat = x[:, _NDENSE:].reshape(-1)
  emb_rows = _sc_gather(emb_table, xs_flat)
  emb = emb_rows.reshape(_B, _NSPARSE * _EMB)
  return _tc_mlp(
      dense, emb,
      bot_w0, bot_b0.reshape(1, -1), bot_w1, bot_b1.reshape(1, -1),
      bot_w2, bot_b2.reshape(1, -1),
      top_w0, top_b0.reshape(1, -1), top_w1, top_b1.reshape(1, -1),
      top_w2, top_b2.reshape(1, -1), top_w3, top_b3.reshape(1, -1),
      top_w4, top_b4.reshape(1, -1))
